# baseline (speedup 1.0000x reference)
import functools

import jax
import jax.numpy as jnp
from jax import lax
from jax.experimental import pallas as pl
from jax.experimental.pallas import tpu as pltpu
from jax.experimental.pallas import tpu_sc as plsc

_N = 10000
_NPAD = 10240
_E = 160000
_BLK = 128
_NB = 80
_EPAD = 16 * _NB * _BLK
_ROWS_PER_TILE = _NPAD // 16
_RCH = _ROWS_PER_TILE // _BLK
_BN_SCALE = float((1.0 + 1e-5) ** -0.5)

_mesh = plsc.VectorSubcoreMesh(core_axis_name="c", subcore_axis_name="s")


@functools.partial(
    pl.kernel,
    out_type=jax.ShapeDtypeStruct((2 * _NPAD, 16), jnp.float32),
    mesh=_mesh,
    scratch_types=[
        pltpu.VMEM_SHARED((_NPAD, 16), jnp.float32),
        pltpu.VMEM((_BLK, 16), jnp.float32),
        pltpu.VMEM((_BLK, 16), jnp.float32),
        pltpu.VMEM((_NB // 2, _BLK), jnp.int32),
    ],
)
def _deg_kernel(dst_hbm, out_hbm, table, ones_v, stage_v, idx_v):
    c = lax.axis_index("c")
    s = lax.axis_index("s")
    base = s * _ROWS_PER_TILE

    def _fill(i, carry):
        ones_v[i] = jnp.ones((16,), jnp.float32)
        stage_v[i] = jnp.zeros((16,), jnp.float32)
        return carry

    lax.fori_loop(0, _BLK, _fill, 0)

    for k in range(_RCH):
        pltpu.sync_copy(stage_v, table.at[pl.ds(base + k * _BLK, _BLK), :])

    pltpu.sync_copy(dst_hbm.at[s, pl.ds(c * (_NB // 2), _NB // 2), :], idx_v)
    plsc.subcore_barrier()

    def _scat(j, carry):
        pltpu.sync_copy(ones_v, table.at[idx_v.at[j]], add=True)
        return carry

    lax.fori_loop(0, _NB // 2, _scat, 0)

    plsc.subcore_barrier()
    pltpu.sync_copy(table.at[pl.ds(base, _ROWS_PER_TILE), :],
                    out_hbm.at[pl.ds(c * _NPAD + base, _ROWS_PER_TILE), :])


_CB = 40
_CBS = 8


def _make_agg(D):
    @functools.partial(
        pl.kernel,
        out_type=jax.ShapeDtypeStruct((2 * _NPAD, D), jnp.float32),
        mesh=_mesh,
        scratch_types=[
            pltpu.VMEM_SHARED((_NPAD, D), jnp.float32),
            pltpu.VMEM((_BLK, D), jnp.float32),
            pltpu.VMEM((_BLK, D), jnp.float32),
            pltpu.VMEM((_CB, _BLK), jnp.int32),
            pltpu.VMEM((_CB, _BLK), jnp.int32),
            pltpu.SemaphoreType.DMA,
            pltpu.SemaphoreType.DMA,
        ],
    )
    def _agg(hs_hbm, src_hbm, dst_hbm, out_hbm,
             table, bufa, bufb, src_v, dst_v, sema, semb):
        c = lax.axis_index("c")
        s = lax.axis_index("s")
        base = s * _ROWS_PER_TILE

        pltpu.sync_copy(hs_hbm.at[pl.ds(c * _NPAD + base, _ROWS_PER_TILE), :],
                        table.at[pl.ds(base, _ROWS_PER_TILE), :])

        plsc.subcore_barrier()

        def _chunk(ch, carry):
            pltpu.sync_copy(src_hbm.at[c, s, pl.ds(ch * _CB, _CB)], src_v)
            pltpu.sync_copy(dst_hbm.at[s, pl.ds(ch * _CB, _CB)], dst_v)
            pltpu.async_copy(hs_hbm.at[src_v.at[0]], bufa, sema)

            def _pair(jj, c2):
                j = jj * 2
                pltpu.async_copy(hs_hbm.at[src_v.at[j + 1]], bufb, semb)
                pltpu.make_async_copy(hs_hbm.at[src_v.at[0]], bufa, sema).wait()
                pltpu.sync_copy(bufa, table.at[dst_v.at[j]], add=True)

                @pl.when(jj < _CB // 2 - 1)
                def _():
                    pltpu.async_copy(hs_hbm.at[src_v.at[j + 2]], bufa, sema)

                pltpu.make_async_copy(hs_hbm.at[src_v.at[0]], bufb, semb).wait()
                pltpu.sync_copy(bufb, table.at[dst_v.at[j + 1]], add=True)
                return c2

            lax.fori_loop(0, _CB // 2, _pair, 0)
            return carry

        lax.fori_loop(0, _NB // _CB, _chunk, 0)

        plsc.subcore_barrier()
        pltpu.sync_copy(table.at[pl.ds(base, _ROWS_PER_TILE), :],
                        out_hbm.at[pl.ds(c * _NPAD + base, _ROWS_PER_TILE), :])

    return _agg


_agg128 = _make_agg(128)


@functools.partial(
    pl.kernel,
    out_type=jax.ShapeDtypeStruct((2 * _NPAD, 128), jnp.float32),
    mesh=_mesh,
    scratch_types=[
        pltpu.VMEM_SHARED((_NPAD, 128), jnp.float32),
        pltpu.VMEM((_BLK, 128), jnp.float32),
        pltpu.VMEM((_BLK, 128), jnp.float32),
        pltpu.VMEM((_CBS, _BLK), jnp.int32),
        pltpu.VMEM((_CBS, _BLK), jnp.int32),
        pltpu.SemaphoreType.DMA,
        pltpu.SemaphoreType.DMA,
    ],
)
def _agg_split(hs_hbm, src_hbm, dst_hbm, zero_hbm, out_hbm,
               table, bufa, bufb, src_v, dst_v, sema, semb):
    c = lax.axis_index("c")
    s = lax.axis_index("s")
    base = s * _ROWS_PER_TILE

    @pl.when(c == 0)
    def _():
        pltpu.sync_copy(hs_hbm.at[pl.ds(base, _ROWS_PER_TILE), :],
                        table.at[pl.ds(base, _ROWS_PER_TILE), :])

    @pl.when(c == 1)
    def _():
        pltpu.sync_copy(zero_hbm, bufa)
        for k in range(_RCH):
            pltpu.sync_copy(bufa, table.at[pl.ds(base + k * _BLK, _BLK), :])

    plsc.subcore_barrier()
    nch = 4 + c * 2

    def _chunk(ch, carry):
        b0 = c * 32 + ch * _CBS
        pltpu.sync_copy(src_hbm.at[s, pl.ds(b0, _CBS)], src_v)
        pltpu.sync_copy(dst_hbm.at[s, pl.ds(b0, _CBS)], dst_v)
        pltpu.async_copy(hs_hbm.at[src_v.at[0]], bufa, sema)

        def _pair(jj, c2):
            j = jj * 2
            pltpu.async_copy(hs_hbm.at[src_v.at[j + 1]], bufb, semb)
            pltpu.make_async_copy(hs_hbm.at[src_v.at[0]], bufa, sema).wait()
            pltpu.sync_copy(bufa, table.at[dst_v.at[j]], add=True)

            @pl.when(jj < _CBS // 2 - 1)
            def _():
                pltpu.async_copy(hs_hbm.at[src_v.at[j + 2]], bufa, sema)

            pltpu.make_async_copy(hs_hbm.at[src_v.at[0]], bufb, semb).wait()
            pltpu.sync_copy(bufb, table.at[dst_v.at[j + 1]], add=True)
            return c2

        lax.fori_loop(0, _CBS // 2, _pair, 0)
        return carry

    lax.fori_loop(0, nch, _chunk, 0)

    plsc.subcore_barrier()
    pltpu.sync_copy(table.at[pl.ds(base, _ROWS_PER_TILE), :],
                    out_hbm.at[pl.ds(c * _NPAD + base, _ROWS_PER_TILE), :])


_R = 1024
_G = _NPAD // _R
_HIGH = lax.Precision.HIGHEST


def _dis_from(deg_ref):
    deg = deg_ref[0, :, 0:1] + deg_ref[1, :, 0:1] + 1.0
    return lax.rsqrt(deg)


def _pre_body(x_ref, deg_ref, w_ref, out_ref):
    dis = _dis_from(deg_ref)
    h = jnp.dot(x_ref[...], w_ref[...],
                preferred_element_type=jnp.float32, precision=_HIGH)
    hs = dis * h
    out_ref[0] = hs[:, :128]
    out_ref[1] = hs[:, 128:]


def _mid_body(agg_ref, deg_ref, b_ref, g_ref, be_ref, w_ref, out_ref, *, split):
    i = pl.program_id(0)
    dis = _dis_from(deg_ref)
    z = jnp.concatenate([agg_ref[0], agg_ref[1]], axis=1) * dis + b_ref[...]
    y = jnp.maximum(g_ref[...] * _BN_SCALE * z + be_ref[...], 0.0)
    h = jnp.dot(y, w_ref[...],
                preferred_element_type=jnp.float32, precision=_HIGH)
    hs = dis * h
    rows = i * _R + lax.broadcasted_iota(jnp.int32, (_R, 1), 0)
    hs = jnp.where(rows < _N, hs, 0.0)
    if split:
        out_ref[0] = hs[:, :128]
        out_ref[1] = hs[:, 128:]
    else:
        out_ref[...] = hs


def _final_body(agg_ref, deg_ref, b_ref, out_ref):
    dis = _dis_from(deg_ref)
    z = (agg_ref[0] + agg_ref[1]) * dis + b_ref[...]
    m = jnp.max(z, axis=1, keepdims=True)
    e = jnp.exp(z - m)
    out_ref[...] = (z - m) - jnp.log(jnp.sum(e, axis=1, keepdims=True))


def _vec_spec(n):
    return pl.BlockSpec((1, n), lambda i: (0, 0))


_pre_call = pl.pallas_call(
    _pre_body,
    grid=(_G,),
    in_specs=[
        pl.BlockSpec((_R, 256), lambda i: (i, 0)),
        pl.BlockSpec((2, _R, 16), lambda i: (0, i, 0)),
        pl.BlockSpec((256, 256), lambda i: (0, 0)),
    ],
    out_specs=pl.BlockSpec((2, _R, 128), lambda i: (0, i, 0)),
    out_shape=jax.ShapeDtypeStruct((2, _NPAD, 128), jnp.float32),
)


def _mid_call(wn, split):
    if split:
        out_specs = pl.BlockSpec((2, _R, 128), lambda i: (0, i, 0))
        out_shape = jax.ShapeDtypeStruct((2, _NPAD, 128), jnp.float32)
    else:
        out_specs = pl.BlockSpec((_R, wn), lambda i: (i, 0))
        out_shape = jax.ShapeDtypeStruct((_NPAD, wn), jnp.float32)
    return pl.pallas_call(
        functools.partial(_mid_body, split=split),
        grid=(_G,),
        in_specs=[
            pl.BlockSpec((2, _R, 128), lambda i: (0, i, 0)),
            pl.BlockSpec((2, _R, 16), lambda i: (0, i, 0)),
            _vec_spec(256), _vec_spec(256), _vec_spec(256),
            pl.BlockSpec((256, wn), lambda i: (0, 0)),
        ],
        out_specs=out_specs,
        out_shape=out_shape,
    )


_mid_256 = _mid_call(256, True)
_mid_128 = _mid_call(128, False)

_final_call = pl.pallas_call(
    _final_body,
    grid=(_G,),
    in_specs=[
        pl.BlockSpec((2, _R, 128), lambda i: (0, i, 0)),
        pl.BlockSpec((2, _R, 16), lambda i: (0, i, 0)),
        _vec_spec(128),
    ],
    out_specs=pl.BlockSpec((_R, 128), lambda i: (i, 0)),
    out_shape=jax.ShapeDtypeStruct((_NPAD, 128), jnp.float32),
)


def kernel(x, edge_index, W1, b1, W2, b2, W3, b3, W4, b4, W5, b5,
           gamma1, beta1, gamma2, beta2, gamma3, beta3, gamma4, beta4):
    src = edge_index[0].astype(jnp.int32)
    dst = edge_index[1].astype(jnp.int32)
    pad = _EPAD - _E
    src_t = jnp.concatenate([src, jnp.full((pad,), _N, jnp.int32)])
    dst_t = jnp.concatenate([dst, jnp.full((pad,), _N, jnp.int32)])
    src_t = src_t.reshape(16, _NB, _BLK)
    dst_t = dst_t.reshape(16, _NB, _BLK)
    src2 = jnp.stack([src_t, src_t + _NPAD])

    deg2 = _deg_kernel(dst_t).reshape(2, _NPAD, 16)

    def layer(hs2, mid, *mid_args):
        agg = _agg128(hs2.reshape(2 * _NPAD, 128), src2, dst_t)
        return mid(agg.reshape(2, _NPAD, 128), deg2, *mid_args)

    xp = jnp.pad(x, ((0, _NPAD - _N), (0, 0)))
    b1r, b2r, b3r, b4r, b5r = (b.reshape(1, -1) for b in (b1, b2, b3, b4, b5))
    g1, g2, g3, g4 = (g.reshape(1, -1) for g in (gamma1, gamma2, gamma3, gamma4))
    be1, be2, be3, be4 = (b.reshape(1, -1) for b in (beta1, beta2, beta3, beta4))

    hs = _pre_call(xp, deg2, W1)
    hs = layer(hs, _mid_256, b1r, g1, be1, W2)
    hs = layer(hs, _mid_256, b2r, g2, be2, W3)
    hs = layer(hs, _mid_256, b3r, g3, be3, W4)
    hs5 = layer(hs, _mid_128, b4r, g4, be4, W5)
    zero128 = jnp.zeros((_BLK, 128), jnp.float32)
    agg5 = _agg_split(hs5, src_t, dst_t, zero128).reshape(2, _NPAD, 128)
    out = _final_call(agg5, deg2, b5r)
    return out[:_N]


# speedup vs baseline: 1.0070x; 1.0070x over previous
import functools

import jax
import jax.numpy as jnp
from jax import lax
from jax.experimental import pallas as pl
from jax.experimental.pallas import tpu as pltpu
from jax.experimental.pallas import tpu_sc as plsc

_N = 10000
_NPAD = 10240
_E = 160000
_BLK = 128
_NB = 80
_EPAD = 16 * _NB * _BLK
_ROWS_PER_TILE = _NPAD // 16
_RCH = _ROWS_PER_TILE // _BLK
_BN_SCALE = float((1.0 + 1e-5) ** -0.5)

_mesh = plsc.VectorSubcoreMesh(core_axis_name="c", subcore_axis_name="s")


@functools.partial(
    pl.kernel,
    out_type=jax.ShapeDtypeStruct((2 * _NPAD, 16), jnp.float32),
    mesh=_mesh,
    scratch_types=[
        pltpu.VMEM_SHARED((_NPAD, 16), jnp.float32),
        pltpu.VMEM((_BLK, 16), jnp.float32),
        pltpu.VMEM((_BLK, 16), jnp.float32),
        pltpu.VMEM((_NB // 2, _BLK), jnp.int32),
    ],
)
def _deg_kernel(dst_hbm, out_hbm, table, ones_v, stage_v, idx_v):
    c = lax.axis_index("c")
    s = lax.axis_index("s")
    base = s * _ROWS_PER_TILE

    def _fill(i, carry):
        ones_v[i] = jnp.ones((16,), jnp.float32)
        stage_v[i] = jnp.zeros((16,), jnp.float32)
        return carry

    lax.fori_loop(0, _BLK, _fill, 0)

    for k in range(_RCH):
        pltpu.sync_copy(stage_v, table.at[pl.ds(base + k * _BLK, _BLK), :])

    pltpu.sync_copy(dst_hbm.at[s, pl.ds(c * (_NB // 2), _NB // 2), :], idx_v)
    plsc.subcore_barrier()

    def _scat(j, carry):
        pltpu.sync_copy(ones_v, table.at[idx_v.at[j]], add=True)
        return carry

    lax.fori_loop(0, _NB // 2, _scat, 0)

    plsc.subcore_barrier()
    pltpu.sync_copy(table.at[pl.ds(base, _ROWS_PER_TILE), :],
                    out_hbm.at[pl.ds(c * _NPAD + base, _ROWS_PER_TILE), :])


_CB = 40
_CBS = 8


def _make_agg(D):
    @functools.partial(
        pl.kernel,
        out_type=jax.ShapeDtypeStruct((2 * _NPAD, D), jnp.float32),
        mesh=_mesh,
        scratch_types=[
            pltpu.VMEM_SHARED((_NPAD, D), jnp.float32),
            pltpu.VMEM((_BLK, D), jnp.float32),
            pltpu.VMEM((_BLK, D), jnp.float32),
            pltpu.VMEM((_CB, _BLK), jnp.int32),
            pltpu.VMEM((_CB, _BLK), jnp.int32),
            pltpu.SemaphoreType.DMA,
            pltpu.SemaphoreType.DMA,
            pltpu.SemaphoreType.DMA,
        ],
    )
    def _agg(hs_hbm, src_hbm, dst_hbm, out_hbm,
             table, bufa, bufb, src_v, dst_v, sema, semb, semi):
        c = lax.axis_index("c")
        s = lax.axis_index("s")
        base = s * _ROWS_PER_TILE

        def _load_idx(ch):
            pltpu.sync_copy(src_hbm.at[c, s, pl.ds(ch * _CB, _CB)], src_v)
            pltpu.sync_copy(dst_hbm.at[s, pl.ds(ch * _CB, _CB)], dst_v)

        _load_idx(0)
        pltpu.async_copy(hs_hbm.at[src_v.at[0]], bufa, sema)
        pltpu.sync_copy(hs_hbm.at[pl.ds(c * _NPAD + base, _ROWS_PER_TILE), :],
                        table.at[pl.ds(base, _ROWS_PER_TILE), :])
        plsc.subcore_barrier()

        for ch in range(_NB // _CB):
            if ch > 0:
                _load_idx(ch)
                pltpu.async_copy(hs_hbm.at[src_v.at[0]], bufa, sema)

            def _pair(jj, c2):
                j = jj * 2
                pltpu.async_copy(hs_hbm.at[src_v.at[j + 1]], bufb, semb)
                pltpu.make_async_copy(hs_hbm.at[src_v.at[0]], bufa, sema).wait()
                pltpu.sync_copy(bufa, table.at[dst_v.at[j]], add=True)

                @pl.when(jj < _CB // 2 - 1)
                def _():
                    pltpu.async_copy(hs_hbm.at[src_v.at[j + 2]], bufa, sema)

                pltpu.make_async_copy(hs_hbm.at[src_v.at[0]], bufb, semb).wait()
                pltpu.sync_copy(bufb, table.at[dst_v.at[j + 1]], add=True)
                return c2

            lax.fori_loop(0, _CB // 2, _pair, 0)

        plsc.subcore_barrier()
        pltpu.sync_copy(table.at[pl.ds(base, _ROWS_PER_TILE), :],
                        out_hbm.at[pl.ds(c * _NPAD + base, _ROWS_PER_TILE), :])

    return _agg


_agg128 = _make_agg(128)


@functools.partial(
    pl.kernel,
    out_type=jax.ShapeDtypeStruct((2 * _NPAD, 128), jnp.float32),
    mesh=_mesh,
    scratch_types=[
        pltpu.VMEM_SHARED((_NPAD, 128), jnp.float32),
        pltpu.VMEM((_BLK, 128), jnp.float32),
        pltpu.VMEM((_BLK, 128), jnp.float32),
        pltpu.VMEM((_CBS, _BLK), jnp.int32),
        pltpu.VMEM((_CBS, _BLK), jnp.int32),
        pltpu.SemaphoreType.DMA,
        pltpu.SemaphoreType.DMA,
    ],
)
def _agg_split(hs_hbm, src_hbm, dst_hbm, zero_hbm, out_hbm,
               table, bufa, bufb, src_v, dst_v, sema, semb):
    c = lax.axis_index("c")
    s = lax.axis_index("s")
    base = s * _ROWS_PER_TILE

    @pl.when(c == 0)
    def _():
        pltpu.sync_copy(hs_hbm.at[pl.ds(base, _ROWS_PER_TILE), :],
                        table.at[pl.ds(base, _ROWS_PER_TILE), :])

    @pl.when(c == 1)
    def _():
        pltpu.sync_copy(zero_hbm, bufa)
        for k in range(_RCH):
            pltpu.sync_copy(bufa, table.at[pl.ds(base + k * _BLK, _BLK), :])

    plsc.subcore_barrier()
    half = _NB // 2

    def _chunk(ch, carry):
        b0 = c * half + ch * _CBS
        pltpu.sync_copy(src_hbm.at[s, pl.ds(b0, _CBS)], src_v)
        pltpu.sync_copy(dst_hbm.at[s, pl.ds(b0, _CBS)], dst_v)
        pltpu.async_copy(hs_hbm.at[src_v.at[0]], bufa, sema)

        def _pair(jj, c2):
            j = jj * 2
            pltpu.async_copy(hs_hbm.at[src_v.at[j + 1]], bufb, semb)
            pltpu.make_async_copy(hs_hbm.at[src_v.at[0]], bufa, sema).wait()
            pltpu.sync_copy(bufa, table.at[dst_v.at[j]], add=True)

            @pl.when(jj < _CBS // 2 - 1)
            def _():
                pltpu.async_copy(hs_hbm.at[src_v.at[j + 2]], bufa, sema)

            pltpu.make_async_copy(hs_hbm.at[src_v.at[0]], bufb, semb).wait()
            pltpu.sync_copy(bufb, table.at[dst_v.at[j + 1]], add=True)
            return c2

        lax.fori_loop(0, _CBS // 2, _pair, 0)
        return carry

    lax.fori_loop(0, half // _CBS, _chunk, 0)

    plsc.subcore_barrier()
    pltpu.sync_copy(table.at[pl.ds(base, _ROWS_PER_TILE), :],
                    out_hbm.at[pl.ds(c * _NPAD + base, _ROWS_PER_TILE), :])


_R = 1024
_G = _NPAD // _R
_HIGH = lax.Precision.HIGHEST


def _dis_from(deg_ref):
    deg = deg_ref[0, :, 0:1] + deg_ref[1, :, 0:1] + 1.0
    return lax.rsqrt(deg)


def _pre_body(x_ref, deg_ref, w_ref, out_ref):
    dis = _dis_from(deg_ref)
    h = jnp.dot(x_ref[...], w_ref[...],
                preferred_element_type=jnp.float32, precision=_HIGH)
    hs = dis * h
    out_ref[0] = hs[:, :128]
    out_ref[1] = hs[:, 128:]


def _mid_body(agg_ref, deg_ref, b_ref, g_ref, be_ref, w_ref, out_ref, *, split):
    i = pl.program_id(0)
    dis = _dis_from(deg_ref)
    z = jnp.concatenate([agg_ref[0], agg_ref[1]], axis=1) * dis + b_ref[...]
    y = jnp.maximum(g_ref[...] * _BN_SCALE * z + be_ref[...], 0.0)
    h = jnp.dot(y, w_ref[...],
                preferred_element_type=jnp.float32, precision=_HIGH)
    hs = dis * h
    rows = i * _R + lax.broadcasted_iota(jnp.int32, (_R, 1), 0)
    hs = jnp.where(rows < _N, hs, 0.0)
    if split:
        out_ref[0] = hs[:, :128]
        out_ref[1] = hs[:, 128:]
    else:
        out_ref[...] = hs


def _final_body(agg_ref, deg_ref, b_ref, out_ref):
    dis = _dis_from(deg_ref)
    z = (agg_ref[0] + agg_ref[1]) * dis + b_ref[...]
    m = jnp.max(z, axis=1, keepdims=True)
    e = jnp.exp(z - m)
    out_ref[...] = (z - m) - jnp.log(jnp.sum(e, axis=1, keepdims=True))


def _vec_spec(n):
    return pl.BlockSpec((1, n), lambda i: (0, 0))


_pre_call = pl.pallas_call(
    _pre_body,
    grid=(_G,),
    in_specs=[
        pl.BlockSpec((_R, 256), lambda i: (i, 0)),
        pl.BlockSpec((2, _R, 16), lambda i: (0, i, 0)),
        pl.BlockSpec((256, 256), lambda i: (0, 0)),
    ],
    out_specs=pl.BlockSpec((2, _R, 128), lambda i: (0, i, 0)),
    out_shape=jax.ShapeDtypeStruct((2, _NPAD, 128), jnp.float32),
)


def _mid_call(wn, split):
    if split:
        out_specs = pl.BlockSpec((2, _R, 128), lambda i: (0, i, 0))
        out_shape = jax.ShapeDtypeStruct((2, _NPAD, 128), jnp.float32)
    else:
        out_specs = pl.BlockSpec((_R, wn), lambda i: (i, 0))
        out_shape = jax.ShapeDtypeStruct((_NPAD, wn), jnp.float32)
    return pl.pallas_call(
        functools.partial(_mid_body, split=split),
        grid=(_G,),
        in_specs=[
            pl.BlockSpec((2, _R, 128), lambda i: (0, i, 0)),
            pl.BlockSpec((2, _R, 16), lambda i: (0, i, 0)),
            _vec_spec(256), _vec_spec(256), _vec_spec(256),
            pl.BlockSpec((256, wn), lambda i: (0, 0)),
        ],
        out_specs=out_specs,
        out_shape=out_shape,
    )


_mid_256 = _mid_call(256, True)
_mid_128 = _mid_call(128, False)

_final_call = pl.pallas_call(
    _final_body,
    grid=(_G,),
    in_specs=[
        pl.BlockSpec((2, _R, 128), lambda i: (0, i, 0)),
        pl.BlockSpec((2, _R, 16), lambda i: (0, i, 0)),
        _vec_spec(128),
    ],
    out_specs=pl.BlockSpec((_R, 128), lambda i: (i, 0)),
    out_shape=jax.ShapeDtypeStruct((_NPAD, 128), jnp.float32),
)


def kernel(x, edge_index, W1, b1, W2, b2, W3, b3, W4, b4, W5, b5,
           gamma1, beta1, gamma2, beta2, gamma3, beta3, gamma4, beta4):
    src = edge_index[0].astype(jnp.int32)
    dst = edge_index[1].astype(jnp.int32)
    pad = _EPAD - _E
    src_t = jnp.concatenate([src, jnp.full((pad,), _N, jnp.int32)])
    dst_t = jnp.concatenate([dst, jnp.full((pad,), _N, jnp.int32)])
    src_t = src_t.reshape(16, _NB, _BLK)
    dst_t = dst_t.reshape(16, _NB, _BLK)
    src2 = jnp.stack([src_t, src_t + _NPAD])

    deg2 = _deg_kernel(dst_t).reshape(2, _NPAD, 16)

    def layer(hs2, mid, *mid_args):
        agg = _agg128(hs2.reshape(2 * _NPAD, 128), src2, dst_t)
        return mid(agg.reshape(2, _NPAD, 128), deg2, *mid_args)

    xp = jnp.pad(x, ((0, _NPAD - _N), (0, 0)))
    b1r, b2r, b3r, b4r, b5r = (b.reshape(1, -1) for b in (b1, b2, b3, b4, b5))
    g1, g2, g3, g4 = (g.reshape(1, -1) for g in (gamma1, gamma2, gamma3, gamma4))
    be1, be2, be3, be4 = (b.reshape(1, -1) for b in (beta1, beta2, beta3, beta4))

    hs = _pre_call(xp, deg2, W1)
    hs = layer(hs, _mid_256, b1r, g1, be1, W2)
    hs = layer(hs, _mid_256, b2r, g2, be2, W3)
    hs = layer(hs, _mid_256, b3r, g3, be3, W4)
    hs5 = layer(hs, _mid_128, b4r, g4, be4, W5)
    zero128 = jnp.zeros((_BLK, 128), jnp.float32)
    agg5 = _agg_split(hs5, src_t, dst_t, zero128).reshape(2, _NPAD, 128)
    out = _final_call(agg5, deg2, b5r)
    return out[:_N]
